# Initial kernel scaffold; baseline (speedup 1.0000x reference)
import jax
import jax.numpy as jnp
from jax.experimental import pallas as pl


def kernel(x, edge_index, batch, W1, b1, W2, b2, W3, b3, Wf1, bf1, Wf2, bf2, eps1, eps2, eps3):
    raise NotImplementedError("write your pallas kernel here")


# speedup vs baseline: 4.4187x; 4.4187x over previous
import functools

import jax
import jax.numpy as jnp
from jax import lax
from jax.experimental import pallas as pl
from jax.experimental.pallas import tpu as pltpu
from jax.experimental.pallas import tpu_sc as plsc

_N = 10000
_E = 320000
_D = 128
_H = 64
_HP = 128
_B = 64

_NC = 2
_NS = 16
_NW = _NC * _NS
_EPW = _E // _NW
_CHUNK = 80
_NCHUNK = _EPW // _CHUNK
_RPS = 632
_RPS_LAST = _N - _RPS * (_NS - 1)


def _sc_scatter_partials(y, src, dst, zeros):
  mesh = plsc.VectorSubcoreMesh(core_axis_name="c", subcore_axis_name="s")

  @functools.partial(
      pl.kernel,
      out_type=jax.ShapeDtypeStruct((_NC, _N, _HP), jnp.float32),
      mesh=mesh,
      scratch_types=[
          pltpu.VMEM((1, _CHUNK), jnp.int32),
          pltpu.VMEM((1, _CHUNK), jnp.int32),
          pltpu.VMEM((1, _CHUNK, _HP), jnp.float32),
          pltpu.VMEM_SHARED((_N, _HP), jnp.float32),
          pltpu.SemaphoreType.DMA,
      ],
  )
  def body(y_hbm, src_hbm, dst_hbm, zeros_hbm, out_hbm, srcv, dstv, rowsv,
           agg_sh, sem):
    c = lax.axis_index("c")
    s = lax.axis_index("s")
    w = c * _NS + s
    @pl.when(s < _NS - 1)
    def _():
      pltpu.sync_copy(zeros_hbm.at[pl.ds(s * _RPS, _RPS)],
                      agg_sh.at[pl.ds(s * _RPS, _RPS)])

    @pl.when(s == _NS - 1)
    def _():
      pltpu.sync_copy(zeros_hbm.at[pl.ds((_NS - 1) * _RPS, _RPS_LAST)],
                      agg_sh.at[pl.ds((_NS - 1) * _RPS, _RPS_LAST)])

    plsc.subcore_barrier()
    base = w * _EPW

    def step(j, carry):
      off = base + j * _CHUNK
      pltpu.sync_copy(src_hbm.at[pl.ds(off, _CHUNK)], srcv.at[0])
      pltpu.sync_copy(dst_hbm.at[pl.ds(off, _CHUNK)], dstv.at[0])
      pltpu.async_copy(y_hbm.at[srcv.at[0]], rowsv.at[0], sem).wait()
      pltpu.sync_copy(rowsv.at[0], agg_sh.at[dstv.at[0]], add=True)
      return carry

    lax.fori_loop(0, _NCHUNK, step, 0)
    plsc.subcore_barrier()

    @pl.when(s < _NS - 1)
    def _():
      pltpu.sync_copy(agg_sh.at[pl.ds(s * _RPS, _RPS)],
                      out_hbm.at[c].at[pl.ds(s * _RPS, _RPS)])

    @pl.when(s == _NS - 1)
    def _():
      pltpu.sync_copy(agg_sh.at[pl.ds((_NS - 1) * _RPS, _RPS_LAST)],
                      out_hbm.at[c].at[pl.ds((_NS - 1) * _RPS, _RPS_LAST)])

  return body(y, src, dst, zeros)


def _mm(x, W):
  def body(x_ref, w_ref, o_ref):
    o_ref[...] = jnp.dot(x_ref[...], w_ref[...],
                         preferred_element_type=jnp.float32)

  return pl.pallas_call(
      body,
      out_shape=jax.ShapeDtypeStruct((x.shape[0], W.shape[1]), jnp.float32),
  )(x, W)


def _combine_mm(y, aggs, W, b2d, scale):
  def body(y_ref, a_ref, w_ref, b_ref, s_ref, o_ref):
    h = s_ref[0, 0] * y_ref[...] + a_ref[0] + a_ref[1] + b_ref[...]
    h = jnp.maximum(h, 0.0)
    o_ref[...] = jnp.dot(h, w_ref[...], preferred_element_type=jnp.float32)

  return pl.pallas_call(
      body,
      out_shape=jax.ShapeDtypeStruct((y.shape[0], W.shape[1]), jnp.float32),
  )(y, aggs, W, b2d, scale)


def _final(y3, aggs, b3_2d, scale3, batch2d, Wf1, bf1_2d, Wf2, bf2_2d):
  def body(y_ref, a_ref, b_ref, s_ref, batch_ref, wf1_ref, bf1_ref, wf2_ref,
           bf2_ref, o_ref):
    h = s_ref[0, 0] * y_ref[...] + a_ref[0] + a_ref[1] + b_ref[...]
    h = jnp.maximum(h, 0.0)
    seg = batch_ref[...]
    mask = (lax.broadcasted_iota(jnp.int32, (_B, _N), 0) == seg
            ).astype(jnp.float32)
    sums = jnp.dot(mask, h, preferred_element_type=jnp.float32)
    counts = jnp.sum(mask, axis=1, keepdims=True)
    pooled = sums / jnp.maximum(counts, 1.0)
    g = jnp.dot(pooled, wf1_ref[...], preferred_element_type=jnp.float32)
    g = jnp.maximum(g + bf1_ref[...], 0.0)
    o_ref[...] = jnp.dot(g, wf2_ref[...],
                         preferred_element_type=jnp.float32) + bf2_ref[...]

  return pl.pallas_call(
      body,
      out_shape=jax.ShapeDtypeStruct((_B, 1), jnp.float32),
  )(y3, aggs, b3_2d, scale3, batch2d, Wf1, bf1_2d, Wf2, bf2_2d)


def kernel(x, edge_index, batch, W1, b1, W2, b2, W3, b3, Wf1, bf1, Wf2, bf2,
           eps1, eps2, eps3):
  src = edge_index[0]
  dst = edge_index[1]
  zeros = jnp.zeros((_N, _HP), jnp.float32)
  s1 = jnp.reshape(1.0 + eps1, (1, 1))
  s2 = jnp.reshape(1.0 + eps2, (1, 1))
  s3 = jnp.reshape(1.0 + eps3, (1, 1))
  pad = _HP - _H
  W1p = jnp.pad(W1, ((0, 0), (0, pad)))
  W2p = jnp.pad(W2, ((0, pad), (0, pad)))
  W3p = jnp.pad(W3, ((0, pad), (0, pad)))
  b1p = jnp.pad(jnp.reshape(b1, (1, _H)), ((0, 0), (0, pad)))
  b2p = jnp.pad(jnp.reshape(b2, (1, _H)), ((0, 0), (0, pad)))
  b3p = jnp.pad(jnp.reshape(b3, (1, _H)), ((0, 0), (0, pad)))
  Wf1p = jnp.pad(Wf1, ((0, pad), (0, 0)))

  y1 = _mm(x, W1p)
  agg1 = _sc_scatter_partials(y1, src, dst, zeros)
  y2 = _combine_mm(y1, agg1, W2p, b1p, s1)
  agg2 = _sc_scatter_partials(y2, src, dst, zeros)
  y3 = _combine_mm(y2, agg2, W3p, b2p, s2)
  agg3 = _sc_scatter_partials(y3, src, dst, zeros)
  out = _final(y3, agg3, b3p, s3,
               jnp.reshape(batch, (1, _N)), Wf1p, jnp.reshape(bf1, (1, 10)),
               Wf2, jnp.reshape(bf2, (1, 1)))
  return out


# speedup vs baseline: 11.8031x; 2.6712x over previous
import functools

import jax
import jax.numpy as jnp
from jax import lax
from jax.experimental import pallas as pl
from jax.experimental.pallas import tpu as pltpu
from jax.experimental.pallas import tpu_sc as plsc

_N = 10000
_E = 320000
_D = 128
_H = 64
_HP = 128
_B = 64

_NC = 2
_NS = 16
_NW = _NC * _NS
_EPW = _E // _NW
_CHUNK = 80
_NCHUNK = _EPW // _CHUNK
_NSETS = 4
_PRE = 2
_NSCAT = _NSETS - _PRE
_ISETS = 8
_IPRE = 4
_RPS = 632
_RPS_LAST = _N - _RPS * (_NS - 1)


def _sc_scatter_partials(y, src, dst, zeros):
  mesh = plsc.VectorSubcoreMesh(core_axis_name="c", subcore_axis_name="s")

  @functools.partial(
      pl.kernel,
      out_type=jax.ShapeDtypeStruct((_NC, _N, _HP), jnp.float32),
      mesh=mesh,
      scratch_types=[
          pltpu.VMEM((_ISETS, _CHUNK), jnp.int32),
          pltpu.VMEM((_ISETS, _CHUNK), jnp.int32),
          pltpu.VMEM((_NSETS, _CHUNK, _HP), jnp.float32),
          pltpu.VMEM_SHARED((_N, _HP), jnp.float32),
          pltpu.SemaphoreType.DMA,
          pltpu.SemaphoreType.DMA,
          pltpu.SemaphoreType.DMA,
      ],
  )
  def body(y_hbm, src_hbm, dst_hbm, zeros_hbm, out_hbm, src_ring, dst_ring,
           rows, agg_sh, isem, gsem, ssem):
    c = lax.axis_index("c")
    s = lax.axis_index("s")
    w = c * _NS + s
    base = w * _EPW
    @pl.when(s < _NS - 1)
    def _():
      pltpu.sync_copy(zeros_hbm.at[pl.ds(s * _RPS, _RPS)],
                      agg_sh.at[pl.ds(s * _RPS, _RPS)])

    @pl.when(s == _NS - 1)
    def _():
      pltpu.sync_copy(zeros_hbm.at[pl.ds((_NS - 1) * _RPS, _RPS_LAST)],
                      agg_sh.at[pl.ds((_NS - 1) * _RPS, _RPS_LAST)])

    plsc.subcore_barrier()

    def fire_idx(k):
      ib = lax.rem(k, _ISETS)
      pltpu.async_copy(src_hbm.at[pl.ds(base + k * _CHUNK, _CHUNK)],
                       src_ring.at[ib], isem)
      pltpu.async_copy(dst_hbm.at[pl.ds(base + k * _CHUNK, _CHUNK)],
                       dst_ring.at[ib], isem)

    def iwait():
      pltpu.make_async_copy(src_hbm.at[pl.ds(0, _CHUNK)], src_ring.at[0],
                            isem).wait()
      pltpu.make_async_copy(src_hbm.at[pl.ds(0, _CHUNK)], dst_ring.at[0],
                            isem).wait()

    def fire_gather(k):
      pltpu.async_copy(y_hbm.at[src_ring.at[lax.rem(k, _ISETS)]],
                       rows.at[lax.rem(k, _NSETS)], gsem)

    def gwait():
      pltpu.make_async_copy(y_hbm.at[pl.ds(0, _CHUNK)], rows.at[0],
                            gsem).wait()

    def swait():
      pltpu.make_async_copy(y_hbm.at[pl.ds(0, _CHUNK)], rows.at[0],
                            ssem).wait()

    for k in range(_IPRE):
      fire_idx(k)
    for k in range(_PRE):
      iwait()
      fire_gather(k)

    def step(j, carry):
      b = lax.rem(j, _NSETS)
      gwait()

      @pl.when(j >= _NSCAT)
      def _():
        swait()

      @pl.when(j < _NCHUNK - _PRE)
      def _():
        iwait()
        fire_gather(j + _PRE)

      @pl.when(j < _NCHUNK - _IPRE)
      def _():
        fire_idx(j + _IPRE)

      pltpu.async_copy(rows.at[b], agg_sh.at[dst_ring.at[lax.rem(j, _ISETS)]],
                       ssem, add=True)
      return carry

    lax.fori_loop(0, _NCHUNK, step, 0)
    for _k in range(_NSCAT):
      swait()
    plsc.subcore_barrier()

    @pl.when(s < _NS - 1)
    def _():
      pltpu.sync_copy(agg_sh.at[pl.ds(s * _RPS, _RPS)],
                      out_hbm.at[c].at[pl.ds(s * _RPS, _RPS)])

    @pl.when(s == _NS - 1)
    def _():
      pltpu.sync_copy(agg_sh.at[pl.ds((_NS - 1) * _RPS, _RPS_LAST)],
                      out_hbm.at[c].at[pl.ds((_NS - 1) * _RPS, _RPS_LAST)])

  return body(y, src, dst, zeros)


def _mm(x, W):
  def body(x_ref, w_ref, o_ref):
    o_ref[...] = jnp.dot(x_ref[...], w_ref[...],
                         preferred_element_type=jnp.float32)

  return pl.pallas_call(
      body,
      out_shape=jax.ShapeDtypeStruct((x.shape[0], W.shape[1]), jnp.float32),
  )(x, W)


def _combine_mm(y, aggs, W, b2d, scale):
  def body(y_ref, a_ref, w_ref, b_ref, s_ref, o_ref):
    h = s_ref[0, 0] * y_ref[...] + a_ref[0] + a_ref[1] + b_ref[...]
    h = jnp.maximum(h, 0.0)
    o_ref[...] = jnp.dot(h, w_ref[...], preferred_element_type=jnp.float32)

  return pl.pallas_call(
      body,
      out_shape=jax.ShapeDtypeStruct((y.shape[0], W.shape[1]), jnp.float32),
  )(y, aggs, W, b2d, scale)


def _final(y3, aggs, b3_2d, scale3, batch2d, Wf1, bf1_2d, Wf2, bf2_2d):
  def body(y_ref, a_ref, b_ref, s_ref, batch_ref, wf1_ref, bf1_ref, wf2_ref,
           bf2_ref, o_ref):
    h = s_ref[0, 0] * y_ref[...] + a_ref[0] + a_ref[1] + b_ref[...]
    h = jnp.maximum(h, 0.0)
    seg = batch_ref[...]
    mask = (lax.broadcasted_iota(jnp.int32, (_B, _N), 0) == seg
            ).astype(jnp.float32)
    sums = jnp.dot(mask, h, preferred_element_type=jnp.float32)
    counts = jnp.sum(mask, axis=1, keepdims=True)
    pooled = sums / jnp.maximum(counts, 1.0)
    g = jnp.dot(pooled, wf1_ref[...], preferred_element_type=jnp.float32)
    g = jnp.maximum(g + bf1_ref[...], 0.0)
    o_ref[...] = jnp.dot(g, wf2_ref[...],
                         preferred_element_type=jnp.float32) + bf2_ref[...]

  return pl.pallas_call(
      body,
      out_shape=jax.ShapeDtypeStruct((_B, 1), jnp.float32),
  )(y3, aggs, b3_2d, scale3, batch2d, Wf1, bf1_2d, Wf2, bf2_2d)


def kernel(x, edge_index, batch, W1, b1, W2, b2, W3, b3, Wf1, bf1, Wf2, bf2,
           eps1, eps2, eps3):
  src = edge_index[0]
  dst = edge_index[1]
  zeros = jnp.zeros((_N, _HP), jnp.float32)
  s1 = jnp.reshape(1.0 + eps1, (1, 1))
  s2 = jnp.reshape(1.0 + eps2, (1, 1))
  s3 = jnp.reshape(1.0 + eps3, (1, 1))
  pad = _HP - _H
  W1p = jnp.pad(W1, ((0, 0), (0, pad)))
  W2p = jnp.pad(W2, ((0, pad), (0, pad)))
  W3p = jnp.pad(W3, ((0, pad), (0, pad)))
  b1p = jnp.pad(jnp.reshape(b1, (1, _H)), ((0, 0), (0, pad)))
  b2p = jnp.pad(jnp.reshape(b2, (1, _H)), ((0, 0), (0, pad)))
  b3p = jnp.pad(jnp.reshape(b3, (1, _H)), ((0, 0), (0, pad)))
  Wf1p = jnp.pad(Wf1, ((0, pad), (0, 0)))

  y1 = _mm(x, W1p)
  agg1 = _sc_scatter_partials(y1, src, dst, zeros)
  y2 = _combine_mm(y1, agg1, W2p, b1p, s1)
  agg2 = _sc_scatter_partials(y2, src, dst, zeros)
  y3 = _combine_mm(y2, agg2, W3p, b2p, s2)
  agg3 = _sc_scatter_partials(y3, src, dst, zeros)
  out = _final(y3, agg3, b3p, s3,
               jnp.reshape(batch, (1, _N)), Wf1p, jnp.reshape(bf1, (1, 10)),
               Wf2, jnp.reshape(bf2, (1, 1)))
  return out


# speedup vs baseline: 13.6164x; 1.1536x over previous
import functools

import jax
import jax.numpy as jnp
from jax import lax
from jax.experimental import pallas as pl
from jax.experimental.pallas import tpu as pltpu
from jax.experimental.pallas import tpu_sc as plsc

_N = 10000
_E = 320000
_D = 128
_H = 64
_HP = 128
_B = 64

_NC = 2
_NS = 16
_NW = _NC * _NS
_EPW = _E // _NW
_CHUNK = 80
_NCHUNK = _EPW // _CHUNK
_NSETS = 6
_PRE = 3
_NSCAT = _NSETS - _PRE
_ISETS = 8
_IPRE = 4
_RPS = 632
_RPS_LAST = _N - _RPS * (_NS - 1)


def _sc_scatter_partials(y, src, dst, zeros):
  mesh = plsc.VectorSubcoreMesh(core_axis_name="c", subcore_axis_name="s")

  @functools.partial(
      pl.kernel,
      out_type=jax.ShapeDtypeStruct((_NC, _N, _H), jnp.float32),
      mesh=mesh,
      compiler_params=pltpu.CompilerParams(use_tc_tiling_on_sc=False),
      scratch_types=[
          pltpu.VMEM((_ISETS, _CHUNK), jnp.int32),
          pltpu.VMEM((_ISETS, _CHUNK), jnp.int32),
          pltpu.VMEM((_NSETS, _CHUNK, _H), jnp.float32),
          pltpu.VMEM_SHARED((_N, _H), jnp.float32),
          pltpu.SemaphoreType.DMA,
          pltpu.SemaphoreType.DMA,
          pltpu.SemaphoreType.DMA,
      ],
  )
  def body(y_hbm, src_hbm, dst_hbm, zeros_hbm, out_hbm, src_ring, dst_ring,
           rows, agg_sh, isem, gsem, ssem):
    c = lax.axis_index("c")
    s = lax.axis_index("s")
    w = c * _NS + s
    base = w * _EPW
    @pl.when(s < _NS - 1)
    def _():
      pltpu.sync_copy(zeros_hbm.at[pl.ds(s * _RPS, _RPS)],
                      agg_sh.at[pl.ds(s * _RPS, _RPS)])

    @pl.when(s == _NS - 1)
    def _():
      pltpu.sync_copy(zeros_hbm.at[pl.ds((_NS - 1) * _RPS, _RPS_LAST)],
                      agg_sh.at[pl.ds((_NS - 1) * _RPS, _RPS_LAST)])

    plsc.subcore_barrier()

    def fire_idx(k):
      ib = lax.rem(k, _ISETS)
      pltpu.async_copy(src_hbm.at[pl.ds(base + k * _CHUNK, _CHUNK)],
                       src_ring.at[ib], isem)
      pltpu.async_copy(dst_hbm.at[pl.ds(base + k * _CHUNK, _CHUNK)],
                       dst_ring.at[ib], isem)

    def iwait():
      pltpu.make_async_copy(src_hbm.at[pl.ds(0, _CHUNK)], src_ring.at[0],
                            isem).wait()
      pltpu.make_async_copy(src_hbm.at[pl.ds(0, _CHUNK)], dst_ring.at[0],
                            isem).wait()

    def fire_gather(k):
      pltpu.async_copy(y_hbm.at[src_ring.at[lax.rem(k, _ISETS)]],
                       rows.at[lax.rem(k, _NSETS)], gsem)

    def gwait():
      pltpu.make_async_copy(y_hbm.at[pl.ds(0, _CHUNK)], rows.at[0],
                            gsem).wait()

    def swait():
      pltpu.make_async_copy(y_hbm.at[pl.ds(0, _CHUNK)], rows.at[0],
                            ssem).wait()

    for k in range(_IPRE):
      fire_idx(k)
    for k in range(_PRE):
      iwait()
      fire_gather(k)

    def step(j, carry):
      b = lax.rem(j, _NSETS)
      gwait()

      @pl.when(j >= _NSCAT)
      def _():
        swait()

      @pl.when(j < _NCHUNK - _PRE)
      def _():
        iwait()
        fire_gather(j + _PRE)

      @pl.when(j < _NCHUNK - _IPRE)
      def _():
        fire_idx(j + _IPRE)

      pltpu.async_copy(rows.at[b], agg_sh.at[dst_ring.at[lax.rem(j, _ISETS)]],
                       ssem, add=True)
      return carry

    lax.fori_loop(0, _NCHUNK, step, 0)
    for _k in range(_NSCAT):
      swait()
    plsc.subcore_barrier()

    @pl.when(s < _NS - 1)
    def _():
      pltpu.sync_copy(agg_sh.at[pl.ds(s * _RPS, _RPS)],
                      out_hbm.at[c].at[pl.ds(s * _RPS, _RPS)])

    @pl.when(s == _NS - 1)
    def _():
      pltpu.sync_copy(agg_sh.at[pl.ds((_NS - 1) * _RPS, _RPS_LAST)],
                      out_hbm.at[c].at[pl.ds((_NS - 1) * _RPS, _RPS_LAST)])

  return body(y, src, dst, zeros)


def _mm(x, W):
  def body(x_ref, w_ref, o_ref):
    o_ref[...] = jnp.dot(x_ref[...], w_ref[...],
                         preferred_element_type=jnp.float32)

  return pl.pallas_call(
      body,
      out_shape=jax.ShapeDtypeStruct((x.shape[0], W.shape[1]), jnp.float32),
  )(x, W)


def _combine_mm(y, aggs, W, b2d, scale):
  def body(y_ref, a_ref, w_ref, b_ref, s_ref, o_ref):
    h = s_ref[0, 0] * y_ref[...] + a_ref[0] + a_ref[1] + b_ref[...]
    h = jnp.maximum(h, 0.0)
    o_ref[...] = jnp.dot(h, w_ref[...], preferred_element_type=jnp.float32)

  return pl.pallas_call(
      body,
      out_shape=jax.ShapeDtypeStruct((y.shape[0], W.shape[1]), jnp.float32),
  )(y, aggs, W, b2d, scale)


def _final(y3, aggs, b3_2d, scale3, batch2d, Wf1, bf1_2d, Wf2, bf2_2d):
  def body(y_ref, a_ref, b_ref, s_ref, batch_ref, wf1_ref, bf1_ref, wf2_ref,
           bf2_ref, o_ref):
    h = s_ref[0, 0] * y_ref[...] + a_ref[0] + a_ref[1] + b_ref[...]
    h = jnp.maximum(h, 0.0)
    seg = batch_ref[...]
    mask = (lax.broadcasted_iota(jnp.int32, (_B, _N), 0) == seg
            ).astype(jnp.float32)
    sums = jnp.dot(mask, h, preferred_element_type=jnp.float32)
    counts = jnp.sum(mask, axis=1, keepdims=True)
    pooled = sums / jnp.maximum(counts, 1.0)
    g = jnp.dot(pooled, wf1_ref[...], preferred_element_type=jnp.float32)
    g = jnp.maximum(g + bf1_ref[...], 0.0)
    o_ref[...] = jnp.dot(g, wf2_ref[...],
                         preferred_element_type=jnp.float32) + bf2_ref[...]

  return pl.pallas_call(
      body,
      out_shape=jax.ShapeDtypeStruct((_B, 1), jnp.float32),
  )(y3, aggs, b3_2d, scale3, batch2d, Wf1, bf1_2d, Wf2, bf2_2d)


def kernel(x, edge_index, batch, W1, b1, W2, b2, W3, b3, Wf1, bf1, Wf2, bf2,
           eps1, eps2, eps3):
  src = edge_index[0]
  dst = edge_index[1]
  zeros = jnp.zeros((_N, _H), jnp.float32)
  s1 = jnp.reshape(1.0 + eps1, (1, 1))
  s2 = jnp.reshape(1.0 + eps2, (1, 1))
  s3 = jnp.reshape(1.0 + eps3, (1, 1))
  y1 = _mm(x, W1)
  agg1 = _sc_scatter_partials(y1, src, dst, zeros)
  y2 = _combine_mm(y1, agg1, W2, jnp.reshape(b1, (1, _H)), s1)
  agg2 = _sc_scatter_partials(y2, src, dst, zeros)
  y3 = _combine_mm(y2, agg2, W3, jnp.reshape(b2, (1, _H)), s2)
  agg3 = _sc_scatter_partials(y3, src, dst, zeros)
  out = _final(y3, agg3, jnp.reshape(b3, (1, _H)), s3,
               jnp.reshape(batch, (1, _N)), Wf1, jnp.reshape(bf1, (1, 10)),
               Wf2, jnp.reshape(bf2, (1, 1)))
  return out


# speedup vs baseline: 14.2726x; 1.0482x over previous
import functools

import jax
import jax.numpy as jnp
from jax import lax
from jax.experimental import pallas as pl
from jax.experimental.pallas import tpu as pltpu
from jax.experimental.pallas import tpu_sc as plsc

_N = 10000
_E = 320000
_D = 128
_H = 64
_HP = 128
_B = 64

_NC = 2
_NS = 16
_NW = _NC * _NS
_EPW = _E // _NW
_CHUNK = 80
_NCHUNK = _EPW // _CHUNK
_NSETS = 4
_PRE = 2
_NSCAT = _NSETS - _PRE
_ISETS = 8
_IPRE = 4
_RPS = 632
_RPS_LAST = _N - _RPS * (_NS - 1)


def _sc_scatter_partials(y, src, dst, zeros):
  mesh = plsc.VectorSubcoreMesh(core_axis_name="c", subcore_axis_name="s")

  @functools.partial(
      pl.kernel,
      out_type=jax.ShapeDtypeStruct((_NC, _N, _H), jnp.float32),
      mesh=mesh,
      compiler_params=pltpu.CompilerParams(use_tc_tiling_on_sc=False),
      scratch_types=[
          pltpu.VMEM((_ISETS, _CHUNK), jnp.int32),
          pltpu.VMEM((_ISETS, _CHUNK), jnp.int32),
          pltpu.VMEM((_NSETS, _CHUNK, _H), jnp.float32),
          pltpu.VMEM_SHARED((_N, _H), jnp.float32),
          pltpu.SemaphoreType.DMA,
          pltpu.SemaphoreType.DMA,
          pltpu.SemaphoreType.DMA,
      ],
  )
  def body(y_hbm, src_hbm, dst_hbm, zeros_hbm, out_hbm, src_ring, dst_ring,
           rows, agg_sh, isem, gsem, ssem):
    c = lax.axis_index("c")
    s = lax.axis_index("s")
    w = c * _NS + s
    base = w * _EPW
    @pl.when(s < _NS - 1)
    def _():
      pltpu.sync_copy(zeros_hbm.at[pl.ds(s * _RPS, _RPS)],
                      agg_sh.at[pl.ds(s * _RPS, _RPS)])

    @pl.when(s == _NS - 1)
    def _():
      pltpu.sync_copy(zeros_hbm.at[pl.ds((_NS - 1) * _RPS, _RPS_LAST)],
                      agg_sh.at[pl.ds((_NS - 1) * _RPS, _RPS_LAST)])

    plsc.subcore_barrier()

    def fire_idx(k):
      ib = lax.rem(k, _ISETS)
      pltpu.async_copy(src_hbm.at[pl.ds(base + k * _CHUNK, _CHUNK)],
                       src_ring.at[ib], isem)
      pltpu.async_copy(dst_hbm.at[pl.ds(base + k * _CHUNK, _CHUNK)],
                       dst_ring.at[ib], isem)

    def iwait():
      pltpu.make_async_copy(src_hbm.at[pl.ds(0, _CHUNK)], src_ring.at[0],
                            isem).wait()
      pltpu.make_async_copy(src_hbm.at[pl.ds(0, _CHUNK)], dst_ring.at[0],
                            isem).wait()

    def fire_gather(k):
      pltpu.async_copy(y_hbm.at[src_ring.at[lax.rem(k, _ISETS)]],
                       rows.at[lax.rem(k, _NSETS)], gsem)

    def gwait():
      pltpu.make_async_copy(y_hbm.at[pl.ds(0, _CHUNK)], rows.at[0],
                            gsem).wait()

    def swait():
      pltpu.make_async_copy(y_hbm.at[pl.ds(0, _CHUNK)], rows.at[0],
                            ssem).wait()

    for k in range(_IPRE):
      fire_idx(k)
    for k in range(_PRE):
      iwait()
      fire_gather(k)

    def step(j, carry):
      b = lax.rem(j, _NSETS)
      gwait()

      @pl.when(j >= _NSCAT)
      def _():
        swait()

      @pl.when(j < _NCHUNK - _PRE)
      def _():
        iwait()
        fire_gather(j + _PRE)

      @pl.when(j < _NCHUNK - _IPRE)
      def _():
        fire_idx(j + _IPRE)

      pltpu.async_copy(rows.at[b], agg_sh.at[dst_ring.at[lax.rem(j, _ISETS)]],
                       ssem, add=True)
      return carry

    lax.fori_loop(0, _NCHUNK, step, 0)
    for _k in range(_NSCAT):
      swait()
    plsc.subcore_barrier()

    @pl.when(s < _NS - 1)
    def _():
      pltpu.sync_copy(agg_sh.at[pl.ds(s * _RPS, _RPS)],
                      out_hbm.at[c].at[pl.ds(s * _RPS, _RPS)])

    @pl.when(s == _NS - 1)
    def _():
      pltpu.sync_copy(agg_sh.at[pl.ds((_NS - 1) * _RPS, _RPS_LAST)],
                      out_hbm.at[c].at[pl.ds((_NS - 1) * _RPS, _RPS_LAST)])

  return body(y, src, dst, zeros)


def _mm(x, W):
  def body(x_ref, w_ref, o_ref):
    o_ref[...] = jnp.dot(x_ref[...], w_ref[...],
                         preferred_element_type=jnp.float32)

  return pl.pallas_call(
      body,
      out_shape=jax.ShapeDtypeStruct((x.shape[0], W.shape[1]), jnp.float32),
  )(x, W)


def _combine_mm(y, aggs, W, b2d, scale):
  def body(y_ref, a_ref, w_ref, b_ref, s_ref, o_ref):
    h = s_ref[0, 0] * y_ref[...] + a_ref[0] + a_ref[1] + b_ref[...]
    h = jnp.maximum(h, 0.0)
    o_ref[...] = jnp.dot(h, w_ref[...], preferred_element_type=jnp.float32)

  return pl.pallas_call(
      body,
      out_shape=jax.ShapeDtypeStruct((y.shape[0], W.shape[1]), jnp.float32),
  )(y, aggs, W, b2d, scale)


def _final(y3, aggs, b3_2d, scale3, batch2d, Wf1, bf1_2d, Wf2, bf2_2d):
  def body(y_ref, a_ref, b_ref, s_ref, batch_ref, wf1_ref, bf1_ref, wf2_ref,
           bf2_ref, o_ref):
    h = s_ref[0, 0] * y_ref[...] + a_ref[0] + a_ref[1] + b_ref[...]
    h = jnp.maximum(h, 0.0)
    seg = batch_ref[...]
    mask = (lax.broadcasted_iota(jnp.int32, (_B, _N), 0) == seg
            ).astype(jnp.float32)
    sums = jnp.dot(mask, h, preferred_element_type=jnp.float32)
    counts = jnp.sum(mask, axis=1, keepdims=True)
    pooled = sums / jnp.maximum(counts, 1.0)
    g = jnp.dot(pooled, wf1_ref[...], preferred_element_type=jnp.float32)
    g = jnp.maximum(g + bf1_ref[...], 0.0)
    o_ref[...] = jnp.dot(g, wf2_ref[...],
                         preferred_element_type=jnp.float32) + bf2_ref[...]

  return pl.pallas_call(
      body,
      out_shape=jax.ShapeDtypeStruct((_B, 1), jnp.float32),
  )(y3, aggs, b3_2d, scale3, batch2d, Wf1, bf1_2d, Wf2, bf2_2d)


def kernel(x, edge_index, batch, W1, b1, W2, b2, W3, b3, Wf1, bf1, Wf2, bf2,
           eps1, eps2, eps3):
  src = edge_index[0]
  dst = edge_index[1]
  zeros = jnp.zeros((_N, _H), jnp.float32)
  s1 = jnp.reshape(1.0 + eps1, (1, 1))
  s2 = jnp.reshape(1.0 + eps2, (1, 1))
  s3 = jnp.reshape(1.0 + eps3, (1, 1))
  y1 = _mm(x, W1)
  agg1 = _sc_scatter_partials(y1, src, dst, zeros)
  y2 = _combine_mm(y1, agg1, W2, jnp.reshape(b1, (1, _H)), s1)
  agg2 = _sc_scatter_partials(y2, src, dst, zeros)
  y3 = _combine_mm(y2, agg2, W3, jnp.reshape(b2, (1, _H)), s2)
  agg3 = _sc_scatter_partials(y3, src, dst, zeros)
  out = _final(y3, agg3, jnp.reshape(b3, (1, _H)), s3,
               jnp.reshape(batch, (1, _N)), Wf1, jnp.reshape(bf1, (1, 10)),
               Wf2, jnp.reshape(bf2, (1, 1)))
  return out
